# Initial kernel scaffold; baseline (speedup 1.0000x reference)
import jax
import jax.numpy as jnp
from jax import lax
from jax.experimental import pallas as pl
from jax.experimental.pallas import tpu as pltpu
from jax.experimental.pallas import tpu_sc as plsc

N_NODES = 10000
N_EDGES = 320000
D_IN = 128
D_H0 = 64
D_H1 = 32
N_GRAPHS = 256
N_CLASSES = 10

NC = 2
NS = 16
NW = NC * NS

N_PAD = 10240
RPT = N_PAD // NS
CH = 80
EPW = N_EDGES // NW
NCH = EPW // CH
NBUF = 5
LAG = 2

ROW_BLK = 1000
N_ROW_BLKS = N_NODES // ROW_BLK


def _mesh():
    return plsc.VectorSubcoreMesh(
        core_axis_name="c", subcore_axis_name="s", num_cores=NC, num_subcores=NS
    )


def _worker_id():
    return lax.axis_index("s") * NC + lax.axis_index("c")


def _deg_body(dst2d, zeros1, degp, didx, ones_v, zb, acc, ssem):
    c = lax.axis_index("c")
    s = lax.axis_index("s")
    w = _worker_id()
    pltpu.sync_copy(dst2d.at[pl.ds(w * NCH, NCH)], didx)

    def _ofill(i, _):
        ones_v[pl.ds(i * 16, 16)] = jnp.ones((16,), jnp.float32)
        return 0

    lax.fori_loop(0, CH // 16, _ofill, 0)
    pltpu.sync_copy(zeros1, zb)
    pltpu.sync_copy(zb, acc.at[pl.ds(s * RPT, RPT)])
    plsc.subcore_barrier()
    for b in range(NBUF):
        pltpu.async_copy(ones_v, acc.at[didx.at[b]], ssem.at[b], add=True)

    def _grp(g, _):
        for b in range(NBUF):
            j = NBUF + g * NBUF + b
            pltpu.make_async_copy(ones_v, acc.at[didx.at[0]], ssem.at[b]).wait()
            pltpu.async_copy(ones_v, acc.at[didx.at[j]], ssem.at[b], add=True)
        return 0

    lax.fori_loop(0, NCH // NBUF - 1, _grp, 0)
    for b in range(NBUF):
        pltpu.make_async_copy(ones_v, acc.at[didx.at[0]], ssem.at[b]).wait()
    plsc.subcore_barrier()
    pltpu.sync_copy(acc.at[pl.ds(s * RPT, RPT)], zb)
    pltpu.sync_copy(zb, degp.at[c, pl.ds(s * RPT, RPT)])


def _deg_call(dst2d, zeros1):
    return pl.kernel(
        _deg_body,
        out_type=jax.ShapeDtypeStruct((NC, N_PAD), jnp.float32),
        mesh=_mesh(),
        scratch_types=[
            pltpu.VMEM((NCH, CH), jnp.int32),
            pltpu.VMEM((CH,), jnp.float32),
            pltpu.VMEM((RPT,), jnp.float32),
            pltpu.VMEM_SHARED((N_PAD,), jnp.float32),
            pltpu.SemaphoreType.DMA((NBUF,)),
        ],
    )(dst2d, zeros1)


def _agg_body(src2d, dst2d, h, zeros_d, out, sidx, didx, rows, wb, acc,
              gsem, ssem):
    c = lax.axis_index("c")
    s = lax.axis_index("s")
    w = _worker_id()
    pltpu.sync_copy(src2d.at[pl.ds(w * NCH, NCH)], sidx)
    pltpu.sync_copy(dst2d.at[pl.ds(w * NCH, NCH)], didx)
    pltpu.sync_copy(zeros_d, wb)
    pltpu.sync_copy(wb, acc.at[pl.ds(s * RPT, RPT)])
    plsc.subcore_barrier()

    for b in range(LAG):
        pltpu.async_copy(h.at[sidx.at[b]], rows.at[b], gsem.at[b])

    def _step(j, b):
        pltpu.make_async_copy(h.at[sidx.at[0]], rows.at[b], gsem.at[b]).wait()
        pltpu.async_copy(rows.at[b], acc.at[didx.at[j]], ssem.at[b], add=True)
        bg = (b + LAG) % NBUF

        @pl.when(j + LAG < NCH)
        def _():
            @pl.when(j >= NBUF - LAG)
            def _():
                pltpu.make_async_copy(
                    rows.at[bg], acc.at[didx.at[0]], ssem.at[bg]
                ).wait()

            pltpu.async_copy(h.at[sidx.at[j + LAG]], rows.at[bg], gsem.at[bg])

    def _grp(g, _):
        for b in range(NBUF):
            _step(g * NBUF + b, b)
        return 0

    lax.fori_loop(0, NCH // NBUF, _grp, 0)
    for b in range(NBUF):
        pltpu.make_async_copy(rows.at[b], acc.at[didx.at[0]], ssem.at[b]).wait()
    plsc.subcore_barrier()
    pltpu.sync_copy(acc.at[pl.ds(s * RPT, RPT)], wb)
    pltpu.sync_copy(wb, out.at[c, pl.ds(s * RPT, RPT)])


def _agg_call(src2d, dst2d, h, zeros_d, d):
    return pl.kernel(
        _agg_body,
        out_type=jax.ShapeDtypeStruct((NC, N_PAD, d), jnp.float32),
        mesh=_mesh(),
        scratch_types=[
            pltpu.VMEM((NCH, CH), jnp.int32),
            pltpu.VMEM((NCH, CH), jnp.int32),
            pltpu.VMEM((NBUF, CH, d), jnp.float32),
            pltpu.VMEM((RPT, d), jnp.float32),
            pltpu.VMEM_SHARED((N_PAD, d), jnp.float32),
            pltpu.SemaphoreType.DMA((NBUF,)),
            pltpu.SemaphoreType.DMA((NBUF,)),
        ],
    )(src2d, dst2d, h, zeros_d)


def _tc1_body(degt_ref, x_ref, w0_ref, h0_ref, dis_ref):
    d = degt_ref[...]
    dis = lax.rsqrt(d[:, 0:1] + d[:, 1:2] + 1.0)
    h = jnp.dot(x_ref[...], w0_ref[...], preferred_element_type=jnp.float32)
    h0_ref[...] = h * dis
    dis_ref[...] = dis


def _tc1_call(degt, x, w0):
    return pl.pallas_call(
        _tc1_body,
        grid=(N_ROW_BLKS,),
        in_specs=[
            pl.BlockSpec((ROW_BLK, 2), lambda i: (i, 0)),
            pl.BlockSpec((ROW_BLK, D_IN), lambda i: (i, 0)),
            pl.BlockSpec((D_IN, D_H0), lambda i: (0, 0)),
        ],
        out_specs=[
            pl.BlockSpec((ROW_BLK, D_H0), lambda i: (i, 0)),
            pl.BlockSpec((ROW_BLK, 1), lambda i: (i, 0)),
        ],
        out_shape=[
            jax.ShapeDtypeStruct((N_NODES, D_H0), jnp.float32),
            jax.ShapeDtypeStruct((N_NODES, 1), jnp.float32),
        ],
    )(degt, x, w0)


def _tc2_body(aggp_ref, h0_ref, dis_ref, b0_ref, w1_ref, h1_ref):
    dis = dis_ref[...]
    a = (aggp_ref[0] + aggp_ref[1] + h0_ref[...]) * dis + b0_ref[...]
    act = jnp.maximum(a, 0.0)
    h1_ref[...] = (
        jnp.dot(act, w1_ref[...], preferred_element_type=jnp.float32) * dis
    )


def _tc2_call(aggp, h0, dis, b0r, w1):
    return pl.pallas_call(
        _tc2_body,
        grid=(N_ROW_BLKS,),
        in_specs=[
            pl.BlockSpec((NC, ROW_BLK, D_H0), lambda i: (0, i, 0)),
            pl.BlockSpec((ROW_BLK, D_H0), lambda i: (i, 0)),
            pl.BlockSpec((ROW_BLK, 1), lambda i: (i, 0)),
            pl.BlockSpec((1, D_H0), lambda i: (0, 0)),
            pl.BlockSpec((D_H0, D_H1), lambda i: (0, 0)),
        ],
        out_specs=pl.BlockSpec((ROW_BLK, D_H1), lambda i: (i, 0)),
        out_shape=jax.ShapeDtypeStruct((N_NODES, D_H1), jnp.float32),
    )(aggp, h0, dis, b0r, w1)


def _tc3_body(aggp_ref, h1_ref, dis_ref, b1_ref, ngi_ref, wd_ref, bd_ref,
              out_ref, gsum, gcnt):
    i = pl.program_id(0)

    @pl.when(i == 0)
    def _():
        gsum[...] = jnp.zeros_like(gsum)
        gcnt[...] = jnp.zeros_like(gcnt)

    dis = dis_ref[...]
    a = (aggp_ref[0] + aggp_ref[1] + h1_ref[...]) * dis + b1_ref[...]
    h2 = jnp.maximum(a, 0.0)
    oh = (
        lax.broadcasted_iota(jnp.int32, (N_GRAPHS, ROW_BLK), 0) == ngi_ref[...]
    ).astype(jnp.float32)
    gsum[...] += jnp.dot(oh, h2, preferred_element_type=jnp.float32)
    gcnt[...] += jnp.sum(oh, axis=1, keepdims=True)

    @pl.when(i == N_ROW_BLKS - 1)
    def _():
        g = gsum[...] / jnp.maximum(gcnt[...], 1.0)
        out_ref[...] = (
            jnp.dot(g, wd_ref[...], preferred_element_type=jnp.float32)
            + bd_ref[...]
        )


def _tc3_call(aggp, h1, dis, b1r, ngi_row, wd, bdr):
    return pl.pallas_call(
        _tc3_body,
        grid=(N_ROW_BLKS,),
        in_specs=[
            pl.BlockSpec((NC, ROW_BLK, D_H1), lambda i: (0, i, 0)),
            pl.BlockSpec((ROW_BLK, D_H1), lambda i: (i, 0)),
            pl.BlockSpec((ROW_BLK, 1), lambda i: (i, 0)),
            pl.BlockSpec((1, D_H1), lambda i: (0, 0)),
            pl.BlockSpec((1, ROW_BLK), lambda i: (0, i)),
            pl.BlockSpec((D_H1, N_CLASSES), lambda i: (0, 0)),
            pl.BlockSpec((1, N_CLASSES), lambda i: (0, 0)),
        ],
        out_specs=pl.BlockSpec((N_GRAPHS, N_CLASSES), lambda i: (0, 0)),
        out_shape=jax.ShapeDtypeStruct((N_GRAPHS, N_CLASSES), jnp.float32),
        scratch_shapes=[
            pltpu.VMEM((N_GRAPHS, D_H1), jnp.float32),
            pltpu.VMEM((N_GRAPHS, 1), jnp.float32),
        ],
    )(aggp, h1, dis, b1r, ngi_row, wd, bdr)


def kernel(x, edge_index, node_graph_index, W0, b0, W1, b1, Wd, bd):
    src2d = edge_index[0].astype(jnp.int32).reshape(NW * NCH, CH)
    dst2d = edge_index[1].astype(jnp.int32).reshape(NW * NCH, CH)
    ngi_row = node_graph_index.astype(jnp.int32).reshape(1, N_NODES)
    zeros1 = jnp.zeros((RPT,), jnp.float32)
    zeros64 = jnp.zeros((RPT, D_H0), jnp.float32)
    zeros32 = jnp.zeros((RPT, D_H1), jnp.float32)

    degp = _deg_call(dst2d, zeros1)
    degt = degp.T
    h0p, dis = _tc1_call(degt, x, W0)
    agg0 = _agg_call(src2d, dst2d, h0p, zeros64, D_H0)
    h1p = _tc2_call(agg0, h0p, dis, b0.reshape(1, D_H0), W1)
    agg1 = _agg_call(src2d, dst2d, h1p, zeros32, D_H1)
    logits = _tc3_call(agg1, h1p, dis, b1.reshape(1, D_H1), ngi_row, Wd,
                       bd.reshape(1, N_CLASSES))
    return logits


# speedup vs baseline: 40.8539x; 40.8539x over previous
import jax
import jax.numpy as jnp
from jax import lax
from jax.experimental import pallas as pl
from jax.experimental.pallas import tpu as pltpu
from jax.experimental.pallas import tpu_sc as plsc

N_NODES = 10000
N_EDGES = 320000
D_IN = 128
D_H0 = 64
D_H1 = 32
N_GRAPHS = 256
N_CLASSES = 10

NC = 2
NS = 16
NW = NC * NS

N_PAD = 10240
RPT = N_PAD // NS
CH = 80
EPW = N_EDGES // NW
NCH = EPW // CH
NBUF = 5
LAG = 2

ROW_BLK = 1000
N_ROW_BLKS = N_NODES // ROW_BLK


def _mesh():
    return plsc.VectorSubcoreMesh(
        core_axis_name="c", subcore_axis_name="s", num_cores=NC, num_subcores=NS
    )


_SC_PARAMS = pltpu.CompilerParams(use_tc_tiling_on_sc=False)


def _worker_id():
    return lax.axis_index("s") * NC + lax.axis_index("c")


def _deg_body(dst2d, zeros1, degp, didx, ones_v, zb, acc, ssem):
    c = lax.axis_index("c")
    s = lax.axis_index("s")
    w = _worker_id()
    pltpu.sync_copy(dst2d.at[w], didx)

    def _ofill(i, _):
        ones_v[pl.ds(i * 16, 16)] = jnp.ones((16,), jnp.float32)
        return 0

    lax.fori_loop(0, CH // 16, _ofill, 0)
    pltpu.sync_copy(zeros1, zb)
    pltpu.sync_copy(zb, acc.at[pl.ds(s * RPT, RPT)])
    plsc.subcore_barrier()
    for b in range(NBUF):
        pltpu.async_copy(ones_v, acc.at[didx.at[b]], ssem.at[b], add=True)

    def _grp(g, _):
        for b in range(NBUF):
            j = NBUF + g * NBUF + b
            pltpu.make_async_copy(ones_v, acc.at[didx.at[0]], ssem.at[b]).wait()
            pltpu.async_copy(ones_v, acc.at[didx.at[j]], ssem.at[b], add=True)
        return 0

    lax.fori_loop(0, NCH // NBUF - 1, _grp, 0)
    for b in range(NBUF):
        pltpu.make_async_copy(ones_v, acc.at[didx.at[0]], ssem.at[b]).wait()
    plsc.subcore_barrier()
    pltpu.sync_copy(acc.at[pl.ds(s * RPT, RPT)], zb)
    pltpu.sync_copy(zb, degp.at[c, 0, pl.ds(s * RPT, RPT)])


def _deg_call(dst2d, zeros1):
    return pl.kernel(
        _deg_body,
        out_type=jax.ShapeDtypeStruct((NC, 1, N_PAD), jnp.float32),
        mesh=_mesh(),
        scratch_types=[
            pltpu.VMEM((NCH, CH), jnp.int32),
            pltpu.VMEM((CH,), jnp.float32),
            pltpu.VMEM((RPT,), jnp.float32),
            pltpu.VMEM_SHARED((N_PAD,), jnp.float32),
            pltpu.SemaphoreType.DMA((NBUF,)),
        ],
        compiler_params=_SC_PARAMS,
    )(dst2d, zeros1)


def _agg_body(src2d, dst2d, h, zeros_d, out, sidx, didx, rows, wb, acc,
              gsem, ssem):
    c = lax.axis_index("c")
    s = lax.axis_index("s")
    w = _worker_id()
    pltpu.sync_copy(src2d.at[w], sidx)
    pltpu.sync_copy(dst2d.at[w], didx)
    pltpu.sync_copy(zeros_d, wb)
    pltpu.sync_copy(wb, acc.at[pl.ds(s * RPT, RPT)])
    plsc.subcore_barrier()

    for b in range(LAG):
        pltpu.async_copy(h.at[sidx.at[b]], rows.at[b], gsem.at[b])

    def _step(j, b):
        pltpu.make_async_copy(h.at[sidx.at[0]], rows.at[b], gsem.at[b]).wait()
        pltpu.async_copy(rows.at[b], acc.at[didx.at[j]], ssem.at[b], add=True)
        bg = (b + LAG) % NBUF

        @pl.when(j + LAG < NCH)
        def _():
            @pl.when(j >= NBUF - LAG)
            def _():
                pltpu.make_async_copy(
                    rows.at[bg], acc.at[didx.at[0]], ssem.at[bg]
                ).wait()

            pltpu.async_copy(h.at[sidx.at[j + LAG]], rows.at[bg], gsem.at[bg])

    def _grp(g, _):
        for b in range(NBUF):
            _step(g * NBUF + b, b)
        return 0

    lax.fori_loop(0, NCH // NBUF, _grp, 0)
    for b in range(NBUF):
        pltpu.make_async_copy(rows.at[b], acc.at[didx.at[0]], ssem.at[b]).wait()
    plsc.subcore_barrier()
    pltpu.sync_copy(acc.at[pl.ds(s * RPT, RPT)], wb)
    pltpu.sync_copy(wb, out.at[c, pl.ds(s * RPT, RPT)])


def _agg_call(src2d, dst2d, h, zeros_d, d):
    return pl.kernel(
        _agg_body,
        out_type=jax.ShapeDtypeStruct((NC, N_PAD, d), jnp.float32),
        mesh=_mesh(),
        scratch_types=[
            pltpu.VMEM((NCH, CH), jnp.int32),
            pltpu.VMEM((NCH, CH), jnp.int32),
            pltpu.VMEM((NBUF, CH, d), jnp.float32),
            pltpu.VMEM((RPT, d), jnp.float32),
            pltpu.VMEM_SHARED((N_PAD, d), jnp.float32),
            pltpu.SemaphoreType.DMA((NBUF,)),
            pltpu.SemaphoreType.DMA((NBUF,)),
        ],
        compiler_params=_SC_PARAMS,
    )(src2d, dst2d, h, zeros_d)


def _tc1_body(degt_ref, x_ref, w0_ref, h0_ref, dis_ref):
    d = degt_ref[...]
    dis = lax.rsqrt(d[:, 0:1] + d[:, 1:2] + 1.0)
    h = jnp.dot(x_ref[...], w0_ref[...], preferred_element_type=jnp.float32)
    h0_ref[...] = h * dis
    dis_ref[...] = dis


def _tc1_call(degt, x, w0):
    return pl.pallas_call(
        _tc1_body,
        grid=(N_ROW_BLKS,),
        in_specs=[
            pl.BlockSpec((ROW_BLK, 2), lambda i: (i, 0)),
            pl.BlockSpec((ROW_BLK, D_IN), lambda i: (i, 0)),
            pl.BlockSpec((D_IN, D_H0), lambda i: (0, 0)),
        ],
        out_specs=[
            pl.BlockSpec((ROW_BLK, D_H0), lambda i: (i, 0)),
            pl.BlockSpec((ROW_BLK, 1), lambda i: (i, 0)),
        ],
        out_shape=[
            jax.ShapeDtypeStruct((N_NODES, D_H0), jnp.float32),
            jax.ShapeDtypeStruct((N_NODES, 1), jnp.float32),
        ],
    )(degt, x, w0)


def _tc2_body(aggp_ref, h0_ref, dis_ref, b0_ref, w1_ref, h1_ref):
    dis = dis_ref[...]
    a = (aggp_ref[0] + aggp_ref[1] + h0_ref[...]) * dis + b0_ref[...]
    act = jnp.maximum(a, 0.0)
    h1_ref[...] = (
        jnp.dot(act, w1_ref[...], preferred_element_type=jnp.float32) * dis
    )


def _tc2_call(aggp, h0, dis, b0r, w1):
    return pl.pallas_call(
        _tc2_body,
        grid=(N_ROW_BLKS,),
        in_specs=[
            pl.BlockSpec((NC, ROW_BLK, D_H0), lambda i: (0, i, 0)),
            pl.BlockSpec((ROW_BLK, D_H0), lambda i: (i, 0)),
            pl.BlockSpec((ROW_BLK, 1), lambda i: (i, 0)),
            pl.BlockSpec((1, D_H0), lambda i: (0, 0)),
            pl.BlockSpec((D_H0, D_H1), lambda i: (0, 0)),
        ],
        out_specs=pl.BlockSpec((ROW_BLK, D_H1), lambda i: (i, 0)),
        out_shape=jax.ShapeDtypeStruct((N_NODES, D_H1), jnp.float32),
    )(aggp, h0, dis, b0r, w1)


def _tc3_body(aggp_ref, h1_ref, dis_ref, b1_ref, ngi_ref, wd_ref, bd_ref,
              out_ref, gsum, gcnt):
    i = pl.program_id(0)

    @pl.when(i == 0)
    def _():
        gsum[...] = jnp.zeros_like(gsum)
        gcnt[...] = jnp.zeros_like(gcnt)

    dis = dis_ref[...]
    a = (aggp_ref[0] + aggp_ref[1] + h1_ref[...]) * dis + b1_ref[...]
    h2 = jnp.maximum(a, 0.0)
    oh = (
        lax.broadcasted_iota(jnp.int32, (N_GRAPHS, ROW_BLK), 0) == ngi_ref[0]
    ).astype(jnp.float32)
    gsum[...] += jnp.dot(oh, h2, preferred_element_type=jnp.float32)
    gcnt[...] += jnp.sum(oh, axis=1, keepdims=True)

    @pl.when(i == N_ROW_BLKS - 1)
    def _():
        g = gsum[...] / jnp.maximum(gcnt[...], 1.0)
        out_ref[...] = (
            jnp.dot(g, wd_ref[...], preferred_element_type=jnp.float32)
            + bd_ref[...]
        )


def _tc3_call(aggp, h1, dis, b1r, ngi_row, wd, bdr):
    return pl.pallas_call(
        _tc3_body,
        grid=(N_ROW_BLKS,),
        in_specs=[
            pl.BlockSpec((NC, ROW_BLK, D_H1), lambda i: (0, i, 0)),
            pl.BlockSpec((ROW_BLK, D_H1), lambda i: (i, 0)),
            pl.BlockSpec((ROW_BLK, 1), lambda i: (i, 0)),
            pl.BlockSpec((1, D_H1), lambda i: (0, 0)),
            pl.BlockSpec((1, 1, ROW_BLK), lambda i: (i, 0, 0)),
            pl.BlockSpec((D_H1, N_CLASSES), lambda i: (0, 0)),
            pl.BlockSpec((1, N_CLASSES), lambda i: (0, 0)),
        ],
        out_specs=pl.BlockSpec((N_GRAPHS, N_CLASSES), lambda i: (0, 0)),
        out_shape=jax.ShapeDtypeStruct((N_GRAPHS, N_CLASSES), jnp.float32),
        scratch_shapes=[
            pltpu.VMEM((N_GRAPHS, D_H1), jnp.float32),
            pltpu.VMEM((N_GRAPHS, 1), jnp.float32),
        ],
    )(aggp, h1, dis, b1r, ngi_row, wd, bdr)


def kernel(x, edge_index, node_graph_index, W0, b0, W1, b1, Wd, bd):
    src2d = edge_index[0].astype(jnp.int32).reshape(NW, NCH, CH)
    dst2d = edge_index[1].astype(jnp.int32).reshape(NW, NCH, CH)
    ngi_row = node_graph_index.astype(jnp.int32).reshape(N_ROW_BLKS, 1, ROW_BLK)
    zeros1 = jnp.zeros((RPT,), jnp.float32)
    zeros64 = jnp.zeros((RPT, D_H0), jnp.float32)
    zeros32 = jnp.zeros((RPT, D_H1), jnp.float32)

    degp = _deg_call(dst2d, zeros1)
    degt = degp.reshape(NC, N_PAD).T
    h0p, dis = _tc1_call(degt, x, W0)
    agg0 = _agg_call(src2d, dst2d, h0p, zeros64, D_H0)
    h1p = _tc2_call(agg0, h0p, dis, b0.reshape(1, D_H0), W1)
    agg1 = _agg_call(src2d, dst2d, h1p, zeros32, D_H1)
    logits = _tc3_call(agg1, h1p, dis, b1.reshape(1, D_H1), ngi_row, Wd,
                       bd.reshape(1, N_CLASSES))
    return logits


# speedup vs baseline: 45.4428x; 1.1123x over previous
import jax
import jax.numpy as jnp
from jax import lax
from jax.experimental import pallas as pl
from jax.experimental.pallas import tpu as pltpu
from jax.experimental.pallas import tpu_sc as plsc

N_NODES = 10000
N_EDGES = 320000
D_IN = 128
D_H0 = 64
D_H1 = 32
N_GRAPHS = 256
N_CLASSES = 10

NC = 2
NS = 16
NW = NC * NS

N_PAD = 10240
RPT = N_PAD // NS
CH = 80
EPW = N_EDGES // NW
NCH = EPW // CH
NBUF = 5
LAG = 2
LASTR = N_NODES - (NS - 1) * RPT

ROW_BLK = 2000
N_ROW_BLKS = N_NODES // ROW_BLK


def _mesh():
    return plsc.VectorSubcoreMesh(
        core_axis_name="c", subcore_axis_name="s", num_cores=NC, num_subcores=NS
    )


_SC_PARAMS = pltpu.CompilerParams(use_tc_tiling_on_sc=False)


def _worker_id():
    return lax.axis_index("s") * NC + lax.axis_index("c")


def _deg_body(dst2d, zeros1, degp, didx, ones_v, zb, acc, ssem):
    c = lax.axis_index("c")
    s = lax.axis_index("s")
    w = _worker_id()
    pltpu.sync_copy(dst2d.at[w], didx)

    def _ofill(i, _):
        ones_v[pl.ds(i * 16, 16)] = jnp.ones((16,), jnp.float32)
        return 0

    lax.fori_loop(0, CH // 16, _ofill, 0)
    pltpu.sync_copy(zeros1, zb)
    pltpu.sync_copy(zb, acc.at[pl.ds(s * RPT, RPT)])
    plsc.subcore_barrier()
    for b in range(NBUF):
        pltpu.async_copy(ones_v, acc.at[didx.at[b]], ssem.at[b], add=True)

    def _grp(g, _):
        for b in range(NBUF):
            j = NBUF + g * NBUF + b
            pltpu.make_async_copy(ones_v, acc.at[didx.at[0]], ssem.at[b]).wait()
            pltpu.async_copy(ones_v, acc.at[didx.at[j]], ssem.at[b], add=True)
        return 0

    lax.fori_loop(0, NCH // NBUF - 1, _grp, 0)
    for b in range(NBUF):
        pltpu.make_async_copy(ones_v, acc.at[didx.at[0]], ssem.at[b]).wait()
    plsc.subcore_barrier()
    pltpu.sync_copy(acc.at[pl.ds(s * RPT, RPT)], zb)
    pltpu.sync_copy(zb, degp.at[c, 0, pl.ds(s * RPT, RPT)])


def _deg_call(dst2d, zeros1):
    return pl.kernel(
        _deg_body,
        out_type=jax.ShapeDtypeStruct((NC, 1, N_PAD), jnp.float32),
        mesh=_mesh(),
        scratch_types=[
            pltpu.VMEM((NCH, CH), jnp.int32),
            pltpu.VMEM((CH,), jnp.float32),
            pltpu.VMEM((RPT,), jnp.float32),
            pltpu.VMEM_SHARED((N_PAD,), jnp.float32),
            pltpu.SemaphoreType.DMA((NBUF,)),
        ],
        compiler_params=_SC_PARAMS,
    )(dst2d, zeros1)


def _agg_body(src2d, dst2d, h, zeros_d, out, sidx, didx, rows, wb, acc, hs,
              gsem, ssem, *, stage):
    c = lax.axis_index("c")
    s = lax.axis_index("s")
    w = _worker_id()
    pltpu.sync_copy(src2d.at[w], sidx)
    pltpu.sync_copy(dst2d.at[w], didx)
    if stage:
        @pl.when(s < NS - 1)
        def _():
            pltpu.sync_copy(h.at[pl.ds(s * RPT, RPT)], wb)
            pltpu.sync_copy(wb, hs.at[pl.ds(s * RPT, RPT)])

        @pl.when(s == NS - 1)
        def _():
            pltpu.sync_copy(h.at[pl.ds(N_NODES - LASTR, LASTR)],
                            wb.at[pl.ds(0, LASTR)])
            pltpu.sync_copy(wb.at[pl.ds(0, LASTR)],
                            hs.at[pl.ds(N_NODES - LASTR, LASTR)])

        htab = hs
    else:
        htab = h
    pltpu.sync_copy(zeros_d, wb)
    pltpu.sync_copy(wb, acc.at[pl.ds(s * RPT, RPT)])
    plsc.subcore_barrier()

    for b in range(LAG):
        pltpu.async_copy(htab.at[sidx.at[b]], rows.at[b], gsem.at[b])

    def _step(j, b):
        pltpu.make_async_copy(h.at[sidx.at[0]], rows.at[b], gsem.at[b]).wait()
        pltpu.async_copy(rows.at[b], acc.at[didx.at[j]], ssem.at[b], add=True)
        bg = (b + LAG) % NBUF

        @pl.when(j + LAG < NCH)
        def _():
            @pl.when(j >= NBUF - LAG)
            def _():
                pltpu.make_async_copy(
                    rows.at[bg], acc.at[didx.at[0]], ssem.at[bg]
                ).wait()

            pltpu.async_copy(htab.at[sidx.at[j + LAG]], rows.at[bg],
                             gsem.at[bg])

    def _grp(g, _):
        for b in range(NBUF):
            _step(g * NBUF + b, b)
        return 0

    lax.fori_loop(0, NCH // NBUF, _grp, 0)
    for b in range(NBUF):
        pltpu.make_async_copy(rows.at[b], acc.at[didx.at[0]], ssem.at[b]).wait()
    plsc.subcore_barrier()
    pltpu.sync_copy(acc.at[pl.ds(s * RPT, RPT)], wb)
    pltpu.sync_copy(wb, out.at[c, pl.ds(s * RPT, RPT)])


def _agg_call(src2d, dst2d, h, zeros_d, d, stage):
    import functools as _ft
    return pl.kernel(
        _ft.partial(_agg_body, stage=stage),
        out_type=jax.ShapeDtypeStruct((NC, N_PAD, d), jnp.float32),
        mesh=_mesh(),
        scratch_types=[
            pltpu.VMEM((NCH, CH), jnp.int32),
            pltpu.VMEM((NCH, CH), jnp.int32),
            pltpu.VMEM((NBUF, CH, d), jnp.float32),
            pltpu.VMEM((RPT, d), jnp.float32),
            pltpu.VMEM_SHARED((N_PAD, d), jnp.float32),
            pltpu.VMEM_SHARED((N_PAD, d) if stage else (8, d), jnp.float32),
            pltpu.SemaphoreType.DMA((NBUF,)),
            pltpu.SemaphoreType.DMA((NBUF,)),
        ],
        compiler_params=_SC_PARAMS,
    )(src2d, dst2d, h, zeros_d)


def _tc1_body(degt_ref, x_ref, w0_ref, h0_ref, dis_ref):
    d = degt_ref[...]
    dis = lax.rsqrt(d[:, 0:1] + d[:, 1:2] + 1.0)
    h = jnp.dot(x_ref[...], w0_ref[...], preferred_element_type=jnp.float32)
    h0_ref[...] = h * dis
    dis_ref[...] = dis


def _tc1_call(degt, x, w0):
    return pl.pallas_call(
        _tc1_body,
        grid=(N_ROW_BLKS,),
        in_specs=[
            pl.BlockSpec((ROW_BLK, 2), lambda i: (i, 0)),
            pl.BlockSpec((ROW_BLK, D_IN), lambda i: (i, 0)),
            pl.BlockSpec((D_IN, D_H0), lambda i: (0, 0)),
        ],
        out_specs=[
            pl.BlockSpec((ROW_BLK, D_H0), lambda i: (i, 0)),
            pl.BlockSpec((ROW_BLK, 1), lambda i: (i, 0)),
        ],
        out_shape=[
            jax.ShapeDtypeStruct((N_NODES, D_H0), jnp.float32),
            jax.ShapeDtypeStruct((N_NODES, 1), jnp.float32),
        ],
    )(degt, x, w0)


def _tc2_body(aggp_ref, h0_ref, dis_ref, b0_ref, w1_ref, h1_ref):
    dis = dis_ref[...]
    a = (aggp_ref[0] + aggp_ref[1] + h0_ref[...]) * dis + b0_ref[...]
    act = jnp.maximum(a, 0.0)
    h1_ref[...] = (
        jnp.dot(act, w1_ref[...], preferred_element_type=jnp.float32) * dis
    )


def _tc2_call(aggp, h0, dis, b0r, w1):
    return pl.pallas_call(
        _tc2_body,
        grid=(N_ROW_BLKS,),
        in_specs=[
            pl.BlockSpec((NC, ROW_BLK, D_H0), lambda i: (0, i, 0)),
            pl.BlockSpec((ROW_BLK, D_H0), lambda i: (i, 0)),
            pl.BlockSpec((ROW_BLK, 1), lambda i: (i, 0)),
            pl.BlockSpec((1, D_H0), lambda i: (0, 0)),
            pl.BlockSpec((D_H0, D_H1), lambda i: (0, 0)),
        ],
        out_specs=pl.BlockSpec((ROW_BLK, D_H1), lambda i: (i, 0)),
        out_shape=jax.ShapeDtypeStruct((N_NODES, D_H1), jnp.float32),
    )(aggp, h0, dis, b0r, w1)


def _tc3_body(aggp_ref, h1_ref, dis_ref, b1_ref, ngi_ref, wd_ref, bd_ref,
              out_ref, gsum, gcnt):
    i = pl.program_id(0)

    @pl.when(i == 0)
    def _():
        gsum[...] = jnp.zeros_like(gsum)
        gcnt[...] = jnp.zeros_like(gcnt)

    dis = dis_ref[...]
    a = (aggp_ref[0] + aggp_ref[1] + h1_ref[...]) * dis + b1_ref[...]
    h2 = jnp.maximum(a, 0.0)
    oh = (
        lax.broadcasted_iota(jnp.int32, (N_GRAPHS, ROW_BLK), 0) == ngi_ref[0]
    ).astype(jnp.float32)
    gsum[...] += jnp.dot(oh, h2, preferred_element_type=jnp.float32)
    gcnt[...] += jnp.sum(oh, axis=1, keepdims=True)

    @pl.when(i == N_ROW_BLKS - 1)
    def _():
        g = gsum[...] / jnp.maximum(gcnt[...], 1.0)
        out_ref[...] = (
            jnp.dot(g, wd_ref[...], preferred_element_type=jnp.float32)
            + bd_ref[...]
        )


def _tc3_call(aggp, h1, dis, b1r, ngi_row, wd, bdr):
    return pl.pallas_call(
        _tc3_body,
        grid=(N_ROW_BLKS,),
        in_specs=[
            pl.BlockSpec((NC, ROW_BLK, D_H1), lambda i: (0, i, 0)),
            pl.BlockSpec((ROW_BLK, D_H1), lambda i: (i, 0)),
            pl.BlockSpec((ROW_BLK, 1), lambda i: (i, 0)),
            pl.BlockSpec((1, D_H1), lambda i: (0, 0)),
            pl.BlockSpec((1, 1, ROW_BLK), lambda i: (i, 0, 0)),
            pl.BlockSpec((D_H1, N_CLASSES), lambda i: (0, 0)),
            pl.BlockSpec((1, N_CLASSES), lambda i: (0, 0)),
        ],
        out_specs=pl.BlockSpec((N_GRAPHS, N_CLASSES), lambda i: (0, 0)),
        out_shape=jax.ShapeDtypeStruct((N_GRAPHS, N_CLASSES), jnp.float32),
        scratch_shapes=[
            pltpu.VMEM((N_GRAPHS, D_H1), jnp.float32),
            pltpu.VMEM((N_GRAPHS, 1), jnp.float32),
        ],
    )(aggp, h1, dis, b1r, ngi_row, wd, bdr)


def kernel(x, edge_index, node_graph_index, W0, b0, W1, b1, Wd, bd):
    src2d = edge_index[0].astype(jnp.int32).reshape(NW, NCH, CH)
    dst2d = edge_index[1].astype(jnp.int32).reshape(NW, NCH, CH)
    ngi_row = node_graph_index.astype(jnp.int32).reshape(N_ROW_BLKS, 1, ROW_BLK)
    zeros1 = jnp.zeros((RPT,), jnp.float32)
    zeros64 = jnp.zeros((RPT, D_H0), jnp.float32)
    zeros32 = jnp.zeros((RPT, D_H1), jnp.float32)

    degp = _deg_call(dst2d, zeros1)
    degt = degp.reshape(NC, N_PAD).T
    h0p, dis = _tc1_call(degt, x, W0)
    agg0 = _agg_call(src2d, dst2d, h0p, zeros64, D_H0, False)
    h1p = _tc2_call(agg0, h0p, dis, b0.reshape(1, D_H0), W1)
    agg1 = _agg_call(src2d, dst2d, h1p, zeros32, D_H1, True)
    logits = _tc3_call(agg1, h1p, dis, b1.reshape(1, D_H1), ngi_row, Wd,
                       bd.reshape(1, N_CLASSES))
    return logits


# speedup vs baseline: 67.6057x; 1.4877x over previous
import jax
import jax.numpy as jnp
from jax import lax
from jax.experimental import pallas as pl
from jax.experimental.pallas import tpu as pltpu
from jax.experimental.pallas import tpu_sc as plsc

N_NODES = 10000
N_EDGES = 320000
D_IN = 128
D_H0 = 64
D_H1 = 32
N_GRAPHS = 256
N_CLASSES = 10

NC = 2
NS = 16
NW = NC * NS

N_PAD = 10240
RPT = N_PAD // NS
CH = 128
NCH2 = N_EDGES // CH
KCH = NCH2 // NW
NXTRA = NCH2 - KCH * NW
NBUF = 6
LAG = 2
LASTR = N_NODES - (NS - 1) * RPT

ROW_BLK = 5000
N_ROW_BLKS = N_NODES // ROW_BLK


def _mesh():
    return plsc.VectorSubcoreMesh(
        core_axis_name="c", subcore_axis_name="s", num_cores=NC, num_subcores=NS
    )


_SC_PARAMS = pltpu.CompilerParams(use_tc_tiling_on_sc=False)


def _worker_id():
    return lax.axis_index("s") * NC + lax.axis_index("c")


def _deg_body(edges, zeros1, degp, idxs, xidx, ones_v, zb, acc, ssem):
    c = lax.axis_index("c")
    s = lax.axis_index("s")
    w = _worker_id()
    pltpu.sync_copy(edges.at[pl.ds(w * KCH, KCH)], idxs)

    def _ofill(i, _):
        ones_v[pl.ds(i * 16, 16)] = jnp.ones((16,), jnp.float32)
        return 0

    lax.fori_loop(0, CH // 16, _ofill, 0)
    pltpu.sync_copy(zeros1, zb)
    pltpu.sync_copy(zb, acc.at[pl.ds(s * RPT, RPT)])
    plsc.subcore_barrier()
    for b in range(NBUF):
        pltpu.async_copy(ones_v, acc.at[idxs.at[b, 1]], ssem.at[b], add=True)

    def _grp(g, _):
        for b in range(NBUF):
            j = NBUF + g * NBUF + b
            pltpu.make_async_copy(ones_v, acc.at[idxs.at[0, 1]],
                                  ssem.at[b]).wait()
            pltpu.async_copy(ones_v, acc.at[idxs.at[j, 1]], ssem.at[b],
                             add=True)
        return 0

    lax.fori_loop(0, KCH // NBUF - 1, _grp, 0)
    for b in range(NBUF):
        pltpu.make_async_copy(ones_v, acc.at[idxs.at[0, 1]], ssem.at[b]).wait()

    @pl.when(w < NXTRA)
    def _():
        pltpu.sync_copy(edges.at[NW * KCH + w], xidx)
        pltpu.sync_copy(ones_v, acc.at[xidx.at[1]], add=True)

    plsc.subcore_barrier()
    pltpu.sync_copy(acc.at[pl.ds(s * RPT, RPT)], zb)
    pltpu.sync_copy(zb, degp.at[c, 0, pl.ds(s * RPT, RPT)])


def _deg_call(edges, zeros1):
    return pl.kernel(
        _deg_body,
        out_type=jax.ShapeDtypeStruct((NC, 1, N_PAD), jnp.float32),
        mesh=_mesh(),
        scratch_types=[
            pltpu.VMEM((KCH, 2, CH), jnp.int32),
            pltpu.VMEM((2, CH), jnp.int32),
            pltpu.VMEM((CH,), jnp.float32),
            pltpu.VMEM((RPT,), jnp.float32),
            pltpu.VMEM_SHARED((N_PAD,), jnp.float32),
            pltpu.SemaphoreType.DMA((NBUF,)),
        ],
        compiler_params=_SC_PARAMS,
    )(edges, zeros1)


def _agg_body(edges, h, zeros_d, out, idxs, xidx, rows, wb, acc, hs,
              gsem, ssem):
    c = lax.axis_index("c")
    s = lax.axis_index("s")
    w = _worker_id()
    pltpu.sync_copy(edges.at[pl.ds(w * KCH, KCH)], idxs)
    @pl.when(s < NS - 1)
    def _():
        pltpu.sync_copy(h.at[pl.ds(s * RPT, RPT)], wb)
        pltpu.sync_copy(wb, hs.at[pl.ds(s * RPT, RPT)])

    @pl.when(s == NS - 1)
    def _():
        pltpu.sync_copy(h.at[pl.ds(N_NODES - LASTR, LASTR)],
                        wb.at[pl.ds(0, LASTR)])
        pltpu.sync_copy(wb.at[pl.ds(0, LASTR)],
                        hs.at[pl.ds(N_NODES - LASTR, LASTR)])

    htab = hs
    pltpu.sync_copy(zeros_d, wb)
    pltpu.sync_copy(wb, acc.at[pl.ds(s * RPT, RPT)])
    plsc.subcore_barrier()

    for b in range(LAG):
        pltpu.async_copy(htab.at[idxs.at[b, 0]], rows.at[b], gsem.at[b])

    def _step(j, b):
        pltpu.make_async_copy(h.at[idxs.at[0, 0]], rows.at[b],
                              gsem.at[b]).wait()
        pltpu.async_copy(rows.at[b], acc.at[idxs.at[j, 1]], ssem.at[b],
                         add=True)
        bg = (b + LAG) % NBUF

        @pl.when(j + LAG < KCH)
        def _():
            @pl.when(j >= NBUF - LAG)
            def _():
                pltpu.make_async_copy(
                    rows.at[bg], acc.at[idxs.at[0, 1]], ssem.at[bg]
                ).wait()

            pltpu.async_copy(htab.at[idxs.at[j + LAG, 0]], rows.at[bg],
                             gsem.at[bg])

    def _grp(g, _):
        for b in range(NBUF):
            _step(g * NBUF + b, b)
        return 0

    lax.fori_loop(0, KCH // NBUF, _grp, 0)
    for b in range(NBUF):
        pltpu.make_async_copy(rows.at[b], acc.at[idxs.at[0, 1]],
                              ssem.at[b]).wait()

    @pl.when(w < NXTRA)
    def _():
        pltpu.sync_copy(edges.at[NW * KCH + w], xidx)
        pltpu.async_copy(htab.at[xidx.at[0]], rows.at[0], gsem.at[0]).wait()
        pltpu.sync_copy(rows.at[0], acc.at[xidx.at[1]], add=True)

    plsc.subcore_barrier()
    pltpu.sync_copy(acc.at[pl.ds(s * RPT, RPT)], wb)
    pltpu.sync_copy(wb, out.at[c, pl.ds(s * RPT, RPT)])


def _agg_call(edges, h, zeros_d, d):
    return pl.kernel(
        _agg_body,
        out_type=jax.ShapeDtypeStruct((NC, N_PAD, d), jnp.bfloat16),
        mesh=_mesh(),
        scratch_types=[
            pltpu.VMEM((KCH, 2, CH), jnp.int32),
            pltpu.VMEM((2, CH), jnp.int32),
            pltpu.VMEM((NBUF, CH, d), jnp.bfloat16),
            pltpu.VMEM((RPT, d), jnp.bfloat16),
            pltpu.VMEM_SHARED((N_PAD, d), jnp.bfloat16),
            pltpu.VMEM_SHARED((N_PAD, d), jnp.bfloat16),
            pltpu.SemaphoreType.DMA((NBUF,)),
            pltpu.SemaphoreType.DMA((NBUF,)),
        ],
        compiler_params=_SC_PARAMS,
    )(edges, h, zeros_d)


def _tc1_body(degt_ref, x_ref, w0_ref, h0_ref, dis_ref):
    d = degt_ref[...]
    dis = lax.rsqrt(d[:, 0:1] + d[:, 1:2] + 1.0)
    h = jnp.dot(x_ref[...], w0_ref[...], preferred_element_type=jnp.float32)
    h0_ref[...] = (h * dis).astype(jnp.bfloat16)
    dis_ref[...] = dis


def _tc1_call(degt, x, w0):
    return pl.pallas_call(
        _tc1_body,
        grid=(N_ROW_BLKS,),
        in_specs=[
            pl.BlockSpec((ROW_BLK, 2), lambda i: (i, 0)),
            pl.BlockSpec((ROW_BLK, D_IN), lambda i: (i, 0)),
            pl.BlockSpec((D_IN, D_H0), lambda i: (0, 0)),
        ],
        out_specs=[
            pl.BlockSpec((ROW_BLK, D_H0), lambda i: (i, 0)),
            pl.BlockSpec((ROW_BLK, 1), lambda i: (i, 0)),
        ],
        out_shape=[
            jax.ShapeDtypeStruct((N_NODES, D_H0), jnp.bfloat16),
            jax.ShapeDtypeStruct((N_NODES, 1), jnp.float32),
        ],
    )(degt, x, w0)


def _tc2_body(aggp_ref, h0_ref, dis_ref, b0_ref, w1_ref, h1_ref):
    dis = dis_ref[...]
    agg = (aggp_ref[0].astype(jnp.float32) + aggp_ref[1].astype(jnp.float32)
           + h0_ref[...].astype(jnp.float32))
    a = agg * dis + b0_ref[...]
    act = jnp.maximum(a, 0.0)
    h1_ref[...] = (
        jnp.dot(act, w1_ref[...], preferred_element_type=jnp.float32) * dis
    ).astype(jnp.bfloat16)


def _tc2_call(aggp, h0, dis, b0r, w1):
    return pl.pallas_call(
        _tc2_body,
        grid=(N_ROW_BLKS,),
        in_specs=[
            pl.BlockSpec((NC, ROW_BLK, D_H0), lambda i: (0, i, 0)),
            pl.BlockSpec((ROW_BLK, D_H0), lambda i: (i, 0)),
            pl.BlockSpec((ROW_BLK, 1), lambda i: (i, 0)),
            pl.BlockSpec((1, D_H0), lambda i: (0, 0)),
            pl.BlockSpec((D_H0, D_H1), lambda i: (0, 0)),
        ],
        out_specs=pl.BlockSpec((ROW_BLK, D_H1), lambda i: (i, 0)),
        out_shape=jax.ShapeDtypeStruct((N_NODES, D_H1), jnp.bfloat16),
    )(aggp, h0, dis, b0r, w1)


def _tc3_body(aggp_ref, h1_ref, dis_ref, b1_ref, ngi_ref, wd_ref, bd_ref,
              out_ref, gsum, gcnt):
    i = pl.program_id(0)

    @pl.when(i == 0)
    def _():
        gsum[...] = jnp.zeros_like(gsum)
        gcnt[...] = jnp.zeros_like(gcnt)

    dis = dis_ref[...]
    agg = (aggp_ref[0].astype(jnp.float32) + aggp_ref[1].astype(jnp.float32)
           + h1_ref[...].astype(jnp.float32))
    a = agg * dis + b1_ref[...]
    h2 = jnp.maximum(a, 0.0)
    oh = (
        lax.broadcasted_iota(jnp.int32, (N_GRAPHS, ROW_BLK), 0) == ngi_ref[0]
    ).astype(jnp.float32)
    gsum[...] += jnp.dot(oh, h2, preferred_element_type=jnp.float32)
    gcnt[...] += jnp.sum(oh, axis=1, keepdims=True)

    @pl.when(i == N_ROW_BLKS - 1)
    def _():
        g = gsum[...] / jnp.maximum(gcnt[...], 1.0)
        out_ref[...] = (
            jnp.dot(g, wd_ref[...], preferred_element_type=jnp.float32)
            + bd_ref[...]
        )


def _tc3_call(aggp, h1, dis, b1r, ngi_row, wd, bdr):
    return pl.pallas_call(
        _tc3_body,
        grid=(N_ROW_BLKS,),
        in_specs=[
            pl.BlockSpec((NC, ROW_BLK, D_H1), lambda i: (0, i, 0)),
            pl.BlockSpec((ROW_BLK, D_H1), lambda i: (i, 0)),
            pl.BlockSpec((ROW_BLK, 1), lambda i: (i, 0)),
            pl.BlockSpec((1, D_H1), lambda i: (0, 0)),
            pl.BlockSpec((1, 1, ROW_BLK), lambda i: (i, 0, 0)),
            pl.BlockSpec((D_H1, N_CLASSES), lambda i: (0, 0)),
            pl.BlockSpec((1, N_CLASSES), lambda i: (0, 0)),
        ],
        out_specs=pl.BlockSpec((N_GRAPHS, N_CLASSES), lambda i: (0, 0)),
        out_shape=jax.ShapeDtypeStruct((N_GRAPHS, N_CLASSES), jnp.float32),
        scratch_shapes=[
            pltpu.VMEM((N_GRAPHS, D_H1), jnp.float32),
            pltpu.VMEM((N_GRAPHS, 1), jnp.float32),
        ],
    )(aggp, h1, dis, b1r, ngi_row, wd, bdr)


def kernel(x, edge_index, node_graph_index, W0, b0, W1, b1, Wd, bd):
    edges = (edge_index.astype(jnp.int32)
             .reshape(2, NCH2, CH).transpose(1, 0, 2))
    ngi_row = node_graph_index.astype(jnp.int32).reshape(N_ROW_BLKS, 1, ROW_BLK)
    zeros1 = jnp.zeros((RPT,), jnp.float32)
    zeros64 = jnp.zeros((RPT, D_H0), jnp.bfloat16)
    zeros32 = jnp.zeros((RPT, D_H1), jnp.bfloat16)

    degp = _deg_call(edges, zeros1)
    degt = degp.reshape(NC, N_PAD).T
    h0p, dis = _tc1_call(degt, x, W0)
    agg0 = _agg_call(edges, h0p, zeros64, D_H0)
    h1p = _tc2_call(agg0, h0p, dis, b0.reshape(1, D_H0), W1)
    agg1 = _agg_call(edges, h1p, zeros32, D_H1)
    logits = _tc3_call(agg1, h1p, dis, b1.reshape(1, D_H1), ngi_row, Wd,
                       bd.reshape(1, N_CLASSES))
    return logits
